# baseline (speedup 1.0000x reference)
import functools

import jax
import jax.numpy as jnp
from jax import lax
from jax.experimental import pallas as pl
from jax.experimental.pallas import tpu as pltpu
from jax.experimental.pallas import tpu_sc as plsc

_N = 16384
_D = 256
_C = 128
_BR = 8192
_NB = _N // _BR
_INTRA_W = 1.0
_INTER_W = 0.8
_PREC = jax.lax.Precision.DEFAULT

_NC = 2
_NS = 16
_NW = _NC * _NS
_RPW = _N // _NW
_CHUNK = 128
_NCHUNK = _RPW // _CHUNK


def _sc_segment_sums(features, labels):
    mesh = plsc.VectorSubcoreMesh(core_axis_name="c", subcore_axis_name="s",
                                  num_cores=_NC, num_subcores=_NS)

    @functools.partial(
        pl.kernel,
        mesh=mesh,
        out_type=jax.ShapeDtypeStruct((_NW, _C, _D), jnp.float32),
        scratch_types=[
            pltpu.VMEM((_RPW,), jnp.int32),
            pltpu.SMEM((_RPW,), jnp.int32),
            pltpu.VMEM((_CHUNK, _D), jnp.float32),
            pltpu.VMEM((_C, _D), jnp.float32),
        ],
    )
    def segsum(x_hbm, lab_hbm, sums_out, lab_v, lab_s, rows_v, table_v):
        c = lax.axis_index("c")
        s = lax.axis_index("s")
        wid = c * _NS + s

        pltpu.sync_copy(lab_hbm.at[pl.ds(wid * _RPW, _RPW)], lab_v)

        for r in range(_C):
            for k in range(_D // 16):
                table_v[r, pl.ds(16 * k, 16)] = jnp.zeros((16,), jnp.float32)

        for j in range(_NCHUNK):
            base = wid * _RPW + j * _CHUNK
            pltpu.sync_copy(x_hbm.at[pl.ds(base, _CHUNK)], rows_v)

            def _grp(g, carry, j=j):
                labv = lab_v[pl.ds(j * _CHUNK + g * 16, 16)]
                for t in range(16):
                    lab = labv[t]
                    r = g * 16 + t
                    for k in range(_D // 16):
                        v = rows_v[r, pl.ds(16 * k, 16)]
                        plsc.addupdate(table_v.at[lab, pl.ds(16 * k, 16)], v)
                return carry

            lax.fori_loop(0, _CHUNK // 16, _grp, 0)

        pltpu.sync_copy(table_v, sums_out.at[wid])

    return segsum(features, labels)


def _dot(a, b, dims):
    return jax.lax.dot_general(a, b, (dims, ((), ())),
                               preferred_element_type=jnp.float32,
                               precision=_PREC)


def _tc_body(x_ref, lab_ref, labf_ref, sums_ref, out_ref, cent, minsq, acc):
    i = pl.program_id(0)

    @pl.when(i == 0)
    def _centers_and_inter():
        sums = sums_ref[0]
        for w in range(1, _NW):
            sums = sums + sums_ref[w]
        labf = labf_ref[...].reshape(1, _N)
        cls_full = jax.lax.broadcasted_iota(jnp.int32, (_C, _N), 0)
        onehot_full = (labf == cls_full).astype(jnp.bfloat16)
        counts = _dot(onehot_full, jnp.ones((_N, 1), jnp.bfloat16),
                      ((1,), (0,)))
        cen = sums / jnp.maximum(counts, 1.0)
        cent[...] = cen.astype(jnp.bfloat16)
        csq = cen * cen
        ones_row = jnp.ones((1, _D), jnp.float32)
        cn_col = _dot(csq, ones_row, ((1,), (1,)))
        cn_row = _dot(ones_row, csq, ((1,), (1,)))
        gram = _dot(cen, cen, ((1,), (1,)))
        sq = cn_col + cn_row - 2.0 * gram
        ii = jax.lax.broadcasted_iota(jnp.int32, (_C, _C), 0)
        jj = jax.lax.broadcasted_iota(jnp.int32, (_C, _C), 1)
        off = ii != jj
        minsq[0, 0] = jnp.min(jnp.where(off, sq, jnp.inf))
        acc[0, 0] = 0.0

    lab = lab_ref[...].reshape(1, _BR)
    classes = jax.lax.broadcasted_iota(jnp.int32, (_C, _BR), 0)
    onehot_t = (lab == classes).astype(jnp.bfloat16)

    x = x_ref[...]
    cgath = _dot(onehot_t, cent[...], ((0,), (0,)))
    diff = x - cgath
    d2 = jnp.sum(diff * diff, axis=1)
    acc[0, 0] += jnp.sum(jnp.sqrt(d2))

    @pl.when(i == _NB - 1)
    def _finish():
        intra_loss = acc[0, 0] / _N
        inter_loss = -0.5 * jnp.log(minsq[0, 0])
        loss = _INTRA_W * intra_loss + _INTER_W * inter_loss
        out_ref[...] = loss.reshape(1, 1)


def kernel(features, labels):
    sums32 = _sc_segment_sums(features, labels)
    out = pl.pallas_call(
        _tc_body,
        grid=(_NB,),
        in_specs=[
            pl.BlockSpec((_BR, _D), lambda i: (i, 0)),
            pl.BlockSpec((_BR,), lambda i: (i,)),
            pl.BlockSpec((_N,), lambda i: (0,)),
            pl.BlockSpec((_NW, _C, _D), lambda i: (0, 0, 0)),
        ],
        out_specs=pl.BlockSpec((1, 1), lambda i: (0, 0)),
        out_shape=jax.ShapeDtypeStruct((1, 1), jnp.float32),
        scratch_shapes=[
            pltpu.VMEM((_C, _D), jnp.bfloat16),
            pltpu.SMEM((1, 1), jnp.float32),
            pltpu.SMEM((1, 1), jnp.float32),
        ],
    )(features, labels, labels, sums32)
    return out[0, 0]
